# baseline (speedup 1.0000x reference)
import jax
import jax.numpy as jnp
from jax import lax
from jax.experimental import pallas as pl
from jax.experimental.pallas import tpu as pltpu
from jax.experimental.pallas import tpu_sc as plsc

G = 1024
N = 128
F = 9
H = 64
ROWS_OUT = G * (N + 1)
CH = 64
CIDX = CH * F
OUT_G = (N + 1) * H

_info = plsc.get_sparse_core_info()
NC, NS = _info.num_cores, _info.num_subcores
NW = NC * NS
GPW = G // NW
_ATOM_SPLITS = ((0, 128), (128, 128), (256, 128), (384, 128), (512, 64))


def _sc_body(x_hbm, ind_hbm, outd_hbm, n0_hbm, atom_hbm, idt_hbm, odt_hbm,
             corr_hbm, tok_hbm, out_hbm,
             xi0, ini0, outi0, ci0, ar0, inr0, outr0, cr0, ob0, semi0, semg0,
             xi1, ini1, outi1, ci1, ar1, inr1, outr1, cr1, ob1, semi1, semg1,
             tok):
    c = lax.axis_index("c")
    s = lax.axis_index("s")
    wid = s * NC + c
    pltpu.sync_copy(tok_hbm, tok)
    slot0 = (xi0, ini0, outi0, ci0, ar0, inr0, outr0, cr0, ob0, semi0, semg0)
    slot1 = (xi1, ini1, outi1, ci1, ar1, inr1, outr1, cr1, ob1, semi1, semg1)

    def prefetch_idx(t, slot):
        xi, ini, outi, ci = slot[0], slot[1], slot[2], slot[3]
        semi = slot[9]
        g = wid * GPW + t // 2
        p = t % 2
        xoff = g * (N * F) + p * CIDX
        doff = g * N + p * CH
        pltpu.make_async_copy(x_hbm.at[pl.ds(xoff, CIDX)], xi, semi).start()
        pltpu.make_async_copy(ind_hbm.at[pl.ds(doff, CH)], ini, semi).start()
        pltpu.make_async_copy(outd_hbm.at[pl.ds(doff, CH)], outi, semi).start()
        pltpu.make_async_copy(n0_hbm.at[pl.ds(doff, CH)], ci, semi).start()

    def wait_idx(slot):
        xi, ini, outi, ci = slot[0], slot[1], slot[2], slot[3]
        semi = slot[9]
        pltpu.make_async_copy(x_hbm.at[pl.ds(0, CIDX)], xi, semi).wait()
        pltpu.make_async_copy(ind_hbm.at[pl.ds(0, CH)], ini, semi).wait()
        pltpu.make_async_copy(outd_hbm.at[pl.ds(0, CH)], outi, semi).wait()
        pltpu.make_async_copy(n0_hbm.at[pl.ds(0, CH)], ci, semi).wait()

    def _gather_pairs(slot):
        xi, ini, outi, ci, ar, inr, outr, cr = slot[:8]
        pairs = []
        for off, n in _ATOM_SPLITS:
            pairs.append((atom_hbm.at[xi.at[pl.ds(off, n)]],
                          ar.at[pl.ds(off, n)]))
        pairs.append((idt_hbm.at[ini], inr))
        pairs.append((odt_hbm.at[outi], outr))
        pairs.append((corr_hbm.at[ci], cr))
        return pairs

    def issue_gathers(slot):
        wait_idx(slot)
        semg = slot[10]
        for src, dst in _gather_pairs(slot):
            pltpu.make_async_copy(src, dst, semg).start()

    def wait_gathers(slot):
        semg = slot[10]
        for src, dst in _gather_pairs(slot):
            pltpu.make_async_copy(src, dst, semg).wait()

    def compute_store(t, slot, even):
        ar, inr, outr, cr, ob = slot[4], slot[5], slot[6], slot[7], slot[8]
        g = wid * GPW + t // 2
        base = 1 if even else 0

        def node_body(i, carry2):
            for k in range(H // 16):
                sl = pl.ds(k * 16, 16)
                acc = inr[i, sl] + outr[i, sl]
                acc = acc + cr[i, sl]
                for f in range(F):
                    acc = acc + ar[i * F + f, sl]
                ob[pl.ds((i + base) * H + k * 16, 16)] = acc
            return carry2

        lax.fori_loop(0, CH, node_body, 0)
        if even:
            for k in range(H // 16):
                ob[pl.ds(k * 16, 16)] = tok[pl.ds(k * 16, 16)]
            pltpu.sync_copy(ob, out_hbm.at[pl.ds(g * OUT_G, (CH + 1) * H)])
        else:
            pltpu.sync_copy(
                ob.at[pl.ds(0, CH * H)],
                out_hbm.at[pl.ds(g * OUT_G + (CH + 1) * H, CH * H)])

    prefetch_idx(0, slot0)
    issue_gathers(slot0)
    prefetch_idx(1, slot1)

    def body(j, carry):
        t0 = 2 * j
        t1 = 2 * j + 1
        issue_gathers(slot1)
        wait_gathers(slot0)

        @pl.when(j < GPW - 1)
        def _():
            prefetch_idx(t0 + 2, slot0)

        compute_store(t0, slot0, even=True)

        @pl.when(j < GPW - 1)
        def _():
            issue_gathers(slot0)

        wait_gathers(slot1)

        @pl.when(j < GPW - 1)
        def _():
            prefetch_idx(t1 + 2, slot1)

        compute_store(t1, slot1, even=False)
        return carry

    lax.fori_loop(0, GPW, body, 0)


def _slot_types():
    return [
        pltpu.VMEM((CIDX,), jnp.int32),
        pltpu.VMEM((CH,), jnp.int32),
        pltpu.VMEM((CH,), jnp.int32),
        pltpu.VMEM((CH,), jnp.int32),
        pltpu.VMEM((CIDX, H), jnp.float32),
        pltpu.VMEM((CH, H), jnp.float32),
        pltpu.VMEM((CH, H), jnp.float32),
        pltpu.VMEM((CH, H), jnp.float32),
        pltpu.VMEM(((CH + 1) * H,), jnp.float32),
        pltpu.SemaphoreType.DMA,
        pltpu.SemaphoreType.DMA,
    ]


_sc_call = pl.kernel(
    _sc_body,
    out_type=jax.ShapeDtypeStruct((ROWS_OUT * H,), jnp.float32),
    mesh=plsc.VectorSubcoreMesh(core_axis_name="c", subcore_axis_name="s"),
    compiler_params=pltpu.CompilerParams(use_tc_tiling_on_sc=False),
    scratch_types=_slot_types() + _slot_types() + [
        pltpu.VMEM((H,), jnp.float32),
    ],
)


def kernel(x, in_degree, out_degree, atom_table, in_deg_table, out_deg_table,
           graph_token):
    idt = in_deg_table.at[0].set(0.0)
    odt = out_deg_table.at[0].set(0.0)
    n0 = (x == 0).sum(axis=-1, dtype=jnp.int32).reshape(-1)
    corr = (-jnp.arange(F + 1, dtype=jnp.float32))[:, None] * atom_table[0]
    out = _sc_call(
        x.reshape(-1),
        in_degree.reshape(-1),
        out_degree.reshape(-1),
        n0,
        atom_table, idt, odt, corr,
        graph_token.reshape(-1),
    )
    return out.reshape(G, N + 1, H)


# speedup vs baseline: 6.1693x; 6.1693x over previous
import jax
import jax.numpy as jnp
from jax import lax
from jax.experimental import pallas as pl
from jax.experimental.pallas import tpu as pltpu
from jax.experimental.pallas import tpu_sc as plsc

G = 1024
N = 128
F = 9
H = 64
ROWS_OUT = G * (N + 1)
CH = 64
CIDX = CH * F
OUT_G = (N + 1) * H

_info = plsc.get_sparse_core_info()
NC, NS = _info.num_cores, _info.num_subcores
NW = NC * NS
GPW = G // NW
_ATOM_SPLITS = ((0, 128), (128, 128), (256, 128), (384, 128), (512, 64))


def _sc_body(x_hbm, ind_hbm, outd_hbm, atom_hbm, idt_hbm, odt_hbm, tok_hbm,
             out_hbm,
             xi0, ini0, outi0, ar0, inr0, outr0, ob0, semi0, semg0,
             xi1, ini1, outi1, ar1, inr1, outr1, ob1, semi1, semg1,
             tok):
    c = lax.axis_index("c")
    s = lax.axis_index("s")
    wid = s * NC + c
    pltpu.sync_copy(tok_hbm, tok)
    slot0 = (xi0, ini0, outi0, ar0, inr0, outr0, ob0, semi0, semg0)
    slot1 = (xi1, ini1, outi1, ar1, inr1, outr1, ob1, semi1, semg1)

    def prefetch_idx(t, slot):
        xi, ini, outi = slot[0], slot[1], slot[2]
        semi = slot[7]
        g = wid * GPW + t // 2
        p = t % 2
        xoff = g * (N * F) + p * CIDX
        doff = g * N + p * CH
        pltpu.make_async_copy(x_hbm.at[pl.ds(xoff, CIDX)], xi, semi).start()
        pltpu.make_async_copy(ind_hbm.at[pl.ds(doff, CH)], ini, semi).start()
        pltpu.make_async_copy(outd_hbm.at[pl.ds(doff, CH)], outi, semi).start()

    def wait_idx(slot):
        xi, ini, outi = slot[0], slot[1], slot[2]
        semi = slot[7]
        pltpu.make_async_copy(x_hbm.at[pl.ds(0, CIDX)], xi, semi).wait()
        pltpu.make_async_copy(ind_hbm.at[pl.ds(0, CH)], ini, semi).wait()
        pltpu.make_async_copy(outd_hbm.at[pl.ds(0, CH)], outi, semi).wait()

    def _gather_pairs(slot):
        xi, ini, outi, ar, inr, outr = slot[:6]
        pairs = []
        for off, n in _ATOM_SPLITS:
            pairs.append((atom_hbm.at[xi.at[pl.ds(off, n)]],
                          ar.at[pl.ds(off, n)]))
        pairs.append((idt_hbm.at[ini], inr))
        pairs.append((odt_hbm.at[outi], outr))
        return pairs

    def issue_gathers(slot):
        wait_idx(slot)
        semg = slot[8]
        for src, dst in _gather_pairs(slot):
            pltpu.make_async_copy(src, dst, semg).start()

    def wait_gathers(slot):
        semg = slot[8]
        for src, dst in _gather_pairs(slot):
            pltpu.make_async_copy(src, dst, semg).wait()

    def compute_store(t, slot, even):
        ar, inr, outr, ob = slot[3], slot[4], slot[5], slot[6]
        g = wid * GPW + t // 2
        base = 1 if even else 0

        def node_body(i, carry2):
            for k in range(H // 16):
                sl = pl.ds(k * 16, 16)
                acc = inr[i, sl] + outr[i, sl]
                for f in range(F):
                    acc = acc + ar[i * F + f, sl]
                ob[pl.ds((i + base) * H + k * 16, 16)] = acc
            return carry2

        lax.fori_loop(0, CH, node_body, 0)
        if even:
            for k in range(H // 16):
                ob[pl.ds(k * 16, 16)] = tok[pl.ds(k * 16, 16)]
            pltpu.sync_copy(ob, out_hbm.at[pl.ds(g * OUT_G, (CH + 1) * H)])
        else:
            pltpu.sync_copy(
                ob.at[pl.ds(0, CH * H)],
                out_hbm.at[pl.ds(g * OUT_G + (CH + 1) * H, CH * H)])

    prefetch_idx(0, slot0)
    issue_gathers(slot0)
    prefetch_idx(1, slot1)

    def body(j, carry):
        t0 = 2 * j
        t1 = 2 * j + 1
        issue_gathers(slot1)
        wait_gathers(slot0)

        @pl.when(j < GPW - 1)
        def _():
            prefetch_idx(t0 + 2, slot0)

        compute_store(t0, slot0, even=True)

        @pl.when(j < GPW - 1)
        def _():
            issue_gathers(slot0)

        wait_gathers(slot1)

        @pl.when(j < GPW - 1)
        def _():
            prefetch_idx(t1 + 2, slot1)

        compute_store(t1, slot1, even=False)
        return carry

    lax.fori_loop(0, GPW, body, 0)


def _slot_types():
    return [
        pltpu.VMEM((CIDX,), jnp.int32),
        pltpu.VMEM((CH,), jnp.int32),
        pltpu.VMEM((CH,), jnp.int32),
        pltpu.VMEM((CIDX, H), jnp.float32),
        pltpu.VMEM((CH, H), jnp.float32),
        pltpu.VMEM((CH, H), jnp.float32),
        pltpu.VMEM(((CH + 1) * H,), jnp.float32),
        pltpu.SemaphoreType.DMA,
        pltpu.SemaphoreType.DMA,
    ]


_sc_call = pl.kernel(
    _sc_body,
    out_type=jax.ShapeDtypeStruct((ROWS_OUT * H,), jnp.float32),
    mesh=plsc.VectorSubcoreMesh(core_axis_name="c", subcore_axis_name="s"),
    compiler_params=pltpu.CompilerParams(use_tc_tiling_on_sc=False),
    scratch_types=_slot_types() + _slot_types() + [
        pltpu.VMEM((H,), jnp.float32),
    ],
)


def kernel(x, in_degree, out_degree, atom_table, in_deg_table, out_deg_table,
           graph_token):
    idt = in_deg_table.at[0].set(0.0)
    odt_z = out_deg_table.at[0].set(0.0)
    n0 = (x == 0).sum(axis=-1, dtype=jnp.int32)
    odx = (out_degree + 512 * n0).reshape(-1)
    odt2 = (odt_z[None, :, :]
            - jnp.arange(F + 1, dtype=jnp.float32)[:, None, None]
            * atom_table[0]).reshape((F + 1) * 512, H)
    out = _sc_call(
        x.reshape(-1),
        in_degree.reshape(-1),
        odx,
        atom_table, idt, odt2,
        graph_token.reshape(-1),
    )
    return out.reshape(G, N + 1, H)


# speedup vs baseline: 6.3356x; 1.0270x over previous
import jax
import jax.numpy as jnp
from jax import lax
from jax.experimental import pallas as pl
from jax.experimental.pallas import tpu as pltpu
from jax.experimental.pallas import tpu_sc as plsc

G = 1024
N = 128
F = 9
H = 64
ROWS_OUT = G * (N + 1)
CH = 64
CIDX = CH * F
OUT_G = (N + 1) * H

_info = plsc.get_sparse_core_info()
NC, NS = _info.num_cores, _info.num_subcores
NW = NC * NS
GPW = G // NW
_ATOM_SPLITS = ((0, 128), (128, 128), (256, 128), (384, 128), (512, 64))


def _sc_body(x_hbm, ind_hbm, outd_hbm, atom_hbm, idt_hbm, odt_hbm, tok_hbm,
             out_hbm,
             xi0, ini0, outi0, ar0, inr0, outr0, ob0, semi0, semg0, semo0,
             xi1, ini1, outi1, ar1, inr1, outr1, ob1, semi1, semg1, semo1,
             tok):
    c = lax.axis_index("c")
    s = lax.axis_index("s")
    wid = s * NC + c
    pltpu.sync_copy(tok_hbm, tok)
    slot0 = (xi0, ini0, outi0, ar0, inr0, outr0, ob0, semi0, semg0, semo0)
    slot1 = (xi1, ini1, outi1, ar1, inr1, outr1, ob1, semi1, semg1, semo1)

    def prefetch_idx(t, slot):
        xi, ini, outi = slot[0], slot[1], slot[2]
        semi = slot[7]
        g = wid * GPW + t // 2
        p = t % 2
        xoff = g * (N * F) + p * CIDX
        doff = g * N + p * CH
        pltpu.make_async_copy(x_hbm.at[pl.ds(xoff, CIDX)], xi, semi).start()
        pltpu.make_async_copy(ind_hbm.at[pl.ds(doff, CH)], ini, semi).start()
        pltpu.make_async_copy(outd_hbm.at[pl.ds(doff, CH)], outi, semi).start()

    def wait_idx(slot):
        xi, ini, outi = slot[0], slot[1], slot[2]
        semi = slot[7]
        pltpu.make_async_copy(x_hbm.at[pl.ds(0, CIDX)], xi, semi).wait()
        pltpu.make_async_copy(ind_hbm.at[pl.ds(0, CH)], ini, semi).wait()
        pltpu.make_async_copy(outd_hbm.at[pl.ds(0, CH)], outi, semi).wait()

    def _gather_pairs(slot):
        xi, ini, outi, ar, inr, outr = slot[:6]
        pairs = []
        for off, n in _ATOM_SPLITS:
            pairs.append((atom_hbm.at[xi.at[pl.ds(off, n)]],
                          ar.at[pl.ds(off, n)]))
        pairs.append((idt_hbm.at[ini], inr))
        pairs.append((odt_hbm.at[outi], outr))
        return pairs

    def issue_gathers(slot):
        wait_idx(slot)
        semg = slot[8]
        for src, dst in _gather_pairs(slot):
            pltpu.make_async_copy(src, dst, semg).start()

    def wait_gathers(slot):
        semg = slot[8]
        for src, dst in _gather_pairs(slot):
            pltpu.make_async_copy(src, dst, semg).wait()

    def _store_pair(slot, g, even):
        ob, semo = slot[6], slot[9]
        if even:
            return ob, out_hbm.at[pl.ds(g * OUT_G, (CH + 1) * H)], semo
        return (ob.at[pl.ds(0, CH * H)],
                out_hbm.at[pl.ds(g * OUT_G + (CH + 1) * H, CH * H)], semo)

    def compute_store(t, slot, even, j):
        ar, inr, outr, ob = slot[3], slot[4], slot[5], slot[6]
        g = wid * GPW + t // 2
        base = 1 if even else 0

        @pl.when(j > 0)
        def _():
            src, dst, semo = _store_pair(slot, 0, even)
            pltpu.make_async_copy(src, dst, semo).wait()

        def node_body(i, carry2):
            for k in range(H // 16):
                sl = pl.ds(k * 16, 16)
                acc = inr[i, sl] + outr[i, sl]
                for f in range(F):
                    acc = acc + ar[i * F + f, sl]
                ob[pl.ds((i + base) * H + k * 16, 16)] = acc
            return carry2

        lax.fori_loop(0, CH, node_body, 0)
        src, dst, semo = _store_pair(slot, g, even)
        pltpu.make_async_copy(src, dst, semo).start()

    prefetch_idx(0, slot0)
    issue_gathers(slot0)
    prefetch_idx(1, slot1)
    for k in range(H // 16):
        ob0[pl.ds(k * 16, 16)] = tok[pl.ds(k * 16, 16)]

    def body(j, carry):
        t0 = 2 * j
        t1 = 2 * j + 1
        issue_gathers(slot1)
        wait_gathers(slot0)

        @pl.when(j < GPW - 1)
        def _():
            prefetch_idx(t0 + 2, slot0)

        compute_store(t0, slot0, even=True, j=j)

        @pl.when(j < GPW - 1)
        def _():
            issue_gathers(slot0)

        wait_gathers(slot1)

        @pl.when(j < GPW - 1)
        def _():
            prefetch_idx(t1 + 2, slot1)

        compute_store(t1, slot1, even=False, j=j)
        return carry

    lax.fori_loop(0, GPW, body, 0)
    for slot, even in ((slot0, True), (slot1, False)):
        src, dst, semo = _store_pair(slot, 0, even)
        pltpu.make_async_copy(src, dst, semo).wait()


def _slot_types():
    return [
        pltpu.VMEM((CIDX,), jnp.int32),
        pltpu.VMEM((CH,), jnp.int32),
        pltpu.VMEM((CH,), jnp.int32),
        pltpu.VMEM((CIDX, H), jnp.float32),
        pltpu.VMEM((CH, H), jnp.float32),
        pltpu.VMEM((CH, H), jnp.float32),
        pltpu.VMEM(((CH + 1) * H,), jnp.float32),
        pltpu.SemaphoreType.DMA,
        pltpu.SemaphoreType.DMA,
        pltpu.SemaphoreType.DMA,
    ]


_sc_call = pl.kernel(
    _sc_body,
    out_type=jax.ShapeDtypeStruct((ROWS_OUT * H,), jnp.float32),
    mesh=plsc.VectorSubcoreMesh(core_axis_name="c", subcore_axis_name="s"),
    compiler_params=pltpu.CompilerParams(use_tc_tiling_on_sc=False),
    scratch_types=_slot_types() + _slot_types() + [
        pltpu.VMEM((H,), jnp.float32),
    ],
)


def kernel(x, in_degree, out_degree, atom_table, in_deg_table, out_deg_table,
           graph_token):
    idt = in_deg_table.at[0].set(0.0)
    odt_z = out_deg_table.at[0].set(0.0)
    xf = x.reshape(-1)
    n0 = (xf.reshape(G * N, F) == 0).sum(axis=-1, dtype=jnp.int32)
    odx = out_degree.reshape(-1) + 512 * n0
    odt2 = (odt_z[None, :, :]
            - jnp.arange(F + 1, dtype=jnp.float32)[:, None, None]
            * atom_table[0]).reshape((F + 1) * 512, H)
    out = _sc_call(
        xf,
        in_degree.reshape(-1),
        odx,
        atom_table, idt, odt2,
        graph_token.reshape(-1),
    )
    return out.reshape(G, N + 1, H)


# speedup vs baseline: 6.3402x; 1.0007x over previous
import jax
import jax.numpy as jnp
from jax import lax
from jax.experimental import pallas as pl
from jax.experimental.pallas import tpu as pltpu
from jax.experimental.pallas import tpu_sc as plsc

G = 1024
N = 128
F = 9
H = 64
ROWS_OUT = G * (N + 1)
CH = 64
CIDX = CH * F
OUT_G = (N + 1) * H

_info = plsc.get_sparse_core_info()
NC, NS = _info.num_cores, _info.num_subcores
NW = NC * NS
GPW = G // NW
_ATOM_SPLITS = ((0, 128), (128, 128), (256, 128), (384, 128), (512, 64))


def _sc_body(x_hbm, ind_hbm, outd_hbm, atom_hbm, idt_hbm, odt_hbm, tok_hbm,
             out_hbm,
             xi0, ini0, outi0, ar0, inr0, outr0, ob0, semi0, semg0, semo0,
             xi1, ini1, outi1, ar1, inr1, outr1, ob1, semi1, semg1, semo1,
             tok):
    c = lax.axis_index("c")
    s = lax.axis_index("s")
    wid = s * NC + c
    pltpu.sync_copy(tok_hbm, tok)
    slot0 = (xi0, ini0, outi0, ar0, inr0, outr0, ob0, semi0, semg0, semo0)
    slot1 = (xi1, ini1, outi1, ar1, inr1, outr1, ob1, semi1, semg1, semo1)

    def prefetch_idx(t, slot):
        xi, ini, outi = slot[0], slot[1], slot[2]
        semi = slot[7]
        g = wid * GPW + t // 2
        p = t % 2
        xoff = g * (N * F) + p * CIDX
        doff = g * N + p * CH
        pltpu.make_async_copy(x_hbm.at[pl.ds(xoff, CIDX)], xi, semi).start()
        pltpu.make_async_copy(ind_hbm.at[pl.ds(doff, CH)], ini, semi).start()
        pltpu.make_async_copy(outd_hbm.at[pl.ds(doff, CH)], outi, semi).start()

    def wait_idx(slot):
        xi, ini, outi = slot[0], slot[1], slot[2]
        semi = slot[7]
        pltpu.make_async_copy(x_hbm.at[pl.ds(0, CIDX)], xi, semi).wait()
        pltpu.make_async_copy(ind_hbm.at[pl.ds(0, CH)], ini, semi).wait()
        pltpu.make_async_copy(outd_hbm.at[pl.ds(0, CH)], outi, semi).wait()

    def _gather_pairs(slot):
        xi, ini, outi, ar, inr, outr = slot[:6]
        pairs = []
        for off, n in _ATOM_SPLITS:
            pairs.append((atom_hbm.at[xi.at[pl.ds(off, n)]],
                          ar.at[pl.ds(off, n)]))
        pairs.append((idt_hbm.at[ini], inr))
        pairs.append((odt_hbm.at[outi], outr))
        return pairs

    def issue_gathers(slot):
        wait_idx(slot)
        semg = slot[8]
        for src, dst in _gather_pairs(slot):
            pltpu.make_async_copy(src, dst, semg).start()

    def wait_gathers(slot):
        semg = slot[8]
        for src, dst in _gather_pairs(slot):
            pltpu.make_async_copy(src, dst, semg).wait()

    def _store_pair(slot, g, even):
        ob, semo = slot[6], slot[9]
        if even:
            return ob, out_hbm.at[g, pl.ds(0, CH + 1)], semo
        return (ob.at[pl.ds(0, CH)],
                out_hbm.at[g, pl.ds(CH + 1, CH)], semo)

    def compute_store(t, slot, even, j):
        ar, inr, outr, ob = slot[3], slot[4], slot[5], slot[6]
        g = wid * GPW + t // 2
        base = 1 if even else 0

        @pl.when(j > 0)
        def _():
            src, dst, semo = _store_pair(slot, 0, even)
            pltpu.make_async_copy(src, dst, semo).wait()

        def node_body(i, carry2):
            for k in range(H // 16):
                sl = pl.ds(k * 16, 16)
                acc = inr[i, sl] + outr[i, sl]
                for f in range(F):
                    acc = acc + ar[i * F + f, sl]
                ob[i + base, sl] = acc
            return carry2

        lax.fori_loop(0, CH, node_body, 0)
        src, dst, semo = _store_pair(slot, g, even)
        pltpu.make_async_copy(src, dst, semo).start()

    prefetch_idx(0, slot0)
    issue_gathers(slot0)
    prefetch_idx(1, slot1)
    for k in range(H // 16):
        ob0[0, pl.ds(k * 16, 16)] = tok[pl.ds(k * 16, 16)]

    def body(j, carry):
        t0 = 2 * j
        t1 = 2 * j + 1
        issue_gathers(slot1)
        wait_gathers(slot0)

        @pl.when(j < GPW - 1)
        def _():
            prefetch_idx(t0 + 2, slot0)

        compute_store(t0, slot0, even=True, j=j)

        @pl.when(j < GPW - 1)
        def _():
            issue_gathers(slot0)

        wait_gathers(slot1)

        @pl.when(j < GPW - 1)
        def _():
            prefetch_idx(t1 + 2, slot1)

        compute_store(t1, slot1, even=False, j=j)
        return carry

    lax.fori_loop(0, GPW, body, 0)
    for slot, even in ((slot0, True), (slot1, False)):
        src, dst, semo = _store_pair(slot, 0, even)
        pltpu.make_async_copy(src, dst, semo).wait()


def _slot_types():
    return [
        pltpu.VMEM((CIDX,), jnp.int32),
        pltpu.VMEM((CH,), jnp.int32),
        pltpu.VMEM((CH,), jnp.int32),
        pltpu.VMEM((CIDX, H), jnp.float32),
        pltpu.VMEM((CH, H), jnp.float32),
        pltpu.VMEM((CH, H), jnp.float32),
        pltpu.VMEM((CH + 1, H), jnp.float32),
        pltpu.SemaphoreType.DMA,
        pltpu.SemaphoreType.DMA,
        pltpu.SemaphoreType.DMA,
    ]


_sc_call = pl.kernel(
    _sc_body,
    out_type=jax.ShapeDtypeStruct((G, N + 1, H), jnp.float32),
    mesh=plsc.VectorSubcoreMesh(core_axis_name="c", subcore_axis_name="s"),
    compiler_params=pltpu.CompilerParams(use_tc_tiling_on_sc=False),
    scratch_types=_slot_types() + _slot_types() + [
        pltpu.VMEM((H,), jnp.float32),
    ],
)


def kernel(x, in_degree, out_degree, atom_table, in_deg_table, out_deg_table,
           graph_token):
    idt = in_deg_table.at[0].set(0.0)
    odt_z = out_deg_table.at[0].set(0.0)
    xf = x.reshape(-1)
    n0 = (xf.reshape(G * N, F) == 0).sum(axis=-1, dtype=jnp.int32)
    odx = out_degree.reshape(-1) + 512 * n0
    odt2 = (odt_z[None, :, :]
            - jnp.arange(F + 1, dtype=jnp.float32)[:, None, None]
            * atom_table[0]).reshape((F + 1) * 512, H)
    out = _sc_call(
        xf,
        in_degree.reshape(-1),
        odx,
        atom_table, idt, odt2,
        graph_token.reshape(-1),
    )
    return out
